# baseline (speedup 1.0000x reference)
import functools

import jax
import jax.numpy as jnp
from jax import lax
from jax.experimental import pallas as pl
from jax.experimental.pallas import tpu as pltpu
from jax.experimental.pallas import tpu_sc as plsc

NUM_CORES = 2
NUM_SUBCORES = 16
NUM_WORKERS = NUM_CORES * NUM_SUBCORES

BATCH = 16384
DIM = 32
B_PER_W = BATCH // NUM_WORKERS

_mesh = plsc.VectorSubcoreMesh(core_axis_name="c", subcore_axis_name="s")


@functools.partial(
    pl.kernel,
    mesh=_mesh,
    out_type=jax.ShapeDtypeStruct((BATCH, DIM), jnp.float32),
    compiler_params=pltpu.CompilerParams(use_tc_tiling_on_sc=False),
    scratch_types=[
        pltpu.VMEM((B_PER_W,), jnp.int32),
        pltpu.VMEM((B_PER_W, DIM), jnp.float32),
        pltpu.SemaphoreType.DMA,
    ],
)
def _gather(idx_hbm, table_hbm, out_hbm, idx_v, rows_v, sem):
    wid = lax.axis_index("s") * NUM_CORES + lax.axis_index("c")
    base = wid * B_PER_W
    pltpu.sync_copy(idx_hbm.at[pl.ds(base, B_PER_W)], idx_v)
    pltpu.async_copy(table_hbm.at[idx_v], rows_v, sem).wait()
    pltpu.sync_copy(rows_v, out_hbm.at[pl.ds(base, B_PER_W)])


def kernel(indices, latents):
    return _gather(indices.astype(jnp.int32), latents)


# speedup vs baseline: 3.5781x; 3.5781x over previous
import functools

import jax
import jax.numpy as jnp
from jax import lax
from jax.experimental import pallas as pl
from jax.experimental.pallas import tpu as pltpu
from jax.experimental.pallas import tpu_sc as plsc

NUM_CORES = 2
NUM_SUBCORES = 16
NUM_WORKERS = NUM_CORES * NUM_SUBCORES

BATCH = 16384
DIM = 32
B_PER_W = BATCH // NUM_WORKERS
LANES = 16
BATCH_IDX = 16
N_ROUNDS = B_PER_W // BATCH_IDX

_mesh = plsc.VectorSubcoreMesh(core_axis_name="c", subcore_axis_name="s")


@functools.partial(
    pl.kernel,
    mesh=_mesh,
    out_type=jax.ShapeDtypeStruct((DIM, BATCH), jnp.float32),
    compiler_params=pltpu.CompilerParams(needs_layout_passes=False),
    scratch_types=[
        pltpu.VMEM((B_PER_W,), jnp.int32),
        pltpu.VMEM((BATCH_IDX, DIM, 128), jnp.float32),
        pltpu.VMEM((DIM, B_PER_W), jnp.float32),
        pltpu.SemaphoreType.DMA,
    ],
)
def _gather_t(idx_hbm, table_t_hbm, out_t_hbm, idx_v, chunks_v, cols_v, sem):
    wid = lax.axis_index("s") * NUM_CORES + lax.axis_index("c")
    base = wid * B_PER_W
    pltpu.sync_copy(idx_hbm.at[pl.ds(base, B_PER_W)], idx_v)

    rows_lo = lax.iota(jnp.int32, LANES)
    rows_hi = rows_lo + LANES

    def round_body(b, carry):
        tvec = idx_v[pl.ds(b * BATCH_IDX, BATCH_IDX)]
        g_vec = lax.shift_right_logical(tvec, 7)
        c_vec = lax.bitwise_and(tvec, 127)
        for l in range(BATCH_IDX):
            start = pl.multiple_of(g_vec[l] * 128, 128)
            pltpu.async_copy(
                table_t_hbm.at[:, pl.ds(start, 128)], chunks_v.at[l], sem
            )
        for l in range(BATCH_IDX):
            pltpu.make_async_copy(
                table_t_hbm.at[:, pl.ds(0, 128)], chunks_v.at[l], sem
            ).wait()
        for l in range(BATCH_IDX):
            col = jnp.full((LANES,), c_vec[l], jnp.int32)
            i_col = jnp.full((LANES,), b * BATCH_IDX + l, jnp.int32)
            v_lo = plsc.load_gather(chunks_v.at[l], [rows_lo, col])
            v_hi = plsc.load_gather(chunks_v.at[l], [rows_hi, col])
            plsc.store_scatter(cols_v, [rows_lo, i_col], v_lo)
            plsc.store_scatter(cols_v, [rows_hi, i_col], v_hi)
        return carry

    lax.fori_loop(0, N_ROUNDS, round_body, 0)
    pltpu.sync_copy(cols_v, out_t_hbm.at[:, pl.ds(base, B_PER_W)])


def kernel(indices, latents):
    out_t = _gather_t(indices.astype(jnp.int32), latents.T)
    return out_t.T
